# baseline (speedup 1.0000x reference)
import functools
import math

import jax
import jax.numpy as jnp
from jax import lax
from jax.experimental import pallas as pl
from jax.experimental.pallas import tpu as pltpu
from jax.experimental.pallas import tpu_sc as plsc

D_MODEL = 128
SQRT_D = math.sqrt(D_MODEL)
CB = 128
NIN = 2
NOUT = 4


@functools.cache
def _make_gather(NB: int, L: int, V: int, D: int):
    info = plsc.get_sparse_core_info()
    NC, NS = info.num_cores, info.num_subcores
    NW = NC * NS
    GRP = NOUT
    assert NB == NW * CB and GRP % NIN == 0
    NOUTER = -(-L // GRP)

    mesh = plsc.VectorSubcoreMesh(core_axis_name="c", subcore_axis_name="s")

    @functools.partial(
        pl.kernel,
        out_type=jax.ShapeDtypeStruct((L, NB, D), jnp.float32),
        mesh=mesh,
        scratch_types=[
            pltpu.VMEM((L, CB), jnp.int32),
        ] + [pltpu.VMEM((CB, D), jnp.float32) for _ in range(NIN + NOUT)] + [
            pltpu.SemaphoreType.DMA((NIN,)),
            pltpu.SemaphoreType.DMA((NOUT,)),
        ],
    )
    def gather_kernel(table_hbm, idx_hbm, out_hbm, idx_all, *bufs_and_sems):
        rin = list(bufs_and_sems[:NIN])
        rout = list(bufs_and_sems[NIN:NIN + NOUT])
        gsem, osem = bufs_and_sems[NIN + NOUT], bufs_and_sems[NIN + NOUT + 1]
        wid = lax.axis_index("s") * NC + lax.axis_index("c")

        pltpu.sync_copy(idx_hbm.at[:, wid], idx_all)

        def gather(g, bi):
            return pltpu.make_async_copy(
                table_hbm.at[idx_all.at[g]], rin[bi], gsem.at[bi]
            )

        def out_copy(g, bo):
            return pltpu.make_async_copy(
                rout[bo], out_hbm.at[g, pl.ds(wid * CB, CB)], osem.at[bo]
            )

        gather(0, 0).start()

        def outer(o, _):
            for u in range(GRP):
                g = o * GRP + u
                bi, bo = u % NIN, u % NOUT

                @pl.when(g < L)
                def _():
                    gather(g, bi).wait()

                @pl.when(g + 1 < L)
                def _():
                    gather(g + 1, (u + 1) % NIN).start()

                @pl.when(jnp.logical_and(g >= NOUT, g < L))
                def _():
                    out_copy(g - NOUT, bo).wait()

                @pl.when(g < L)
                def _():
                    def group_body(gr, _):
                        iv = idx_all[g, pl.ds(gr * 16, 16)]
                        scv = jnp.where(iv == 0, 0.0, SQRT_D)
                        for r in range(16):
                            j = gr * 16 + r
                            sc = scv[r]
                            for k in range(D // 16):
                                sl = pl.ds(k * 16, 16)
                                rout[bo][j, sl] = rin[bi][j, sl] * sc
                        return 0

                    lax.fori_loop(0, CB // 16, group_body, 0, unroll=False)
                    out_copy(g, bo).start()
            return 0

        lax.fori_loop(0, NOUTER, outer, 0, unroll=False)

        for g in range(L - NOUT, L):
            out_copy(g, g % NOUT).wait()

    return gather_kernel


def kernel(x, table):
    NB, L = x.shape
    V, D = table.shape
    xt = x.T.reshape(L, NB // CB, CB).astype(jnp.int32)
    out = _make_gather(NB, L, V, D)(table, xt)
    return out.transpose(1, 0, 2)


# speedup vs baseline: 1.2562x; 1.2562x over previous
import functools
import math

import jax
import jax.numpy as jnp
from jax import lax
from jax.experimental import pallas as pl
from jax.experimental.pallas import tpu as pltpu
from jax.experimental.pallas import tpu_sc as plsc

D_MODEL = 128
SQRT_D = math.sqrt(D_MODEL)
NBUF = 5
CB = 128


@functools.cache
def _make_gather(NB: int, L: int, V: int, D: int):
    info = plsc.get_sparse_core_info()
    NC, NS = info.num_cores, info.num_subcores
    NW = NC * NS
    assert NB == NW * CB and L % NBUF == 0

    mesh = plsc.VectorSubcoreMesh(core_axis_name="c", subcore_axis_name="s")

    @functools.partial(
        pl.kernel,
        out_type=jax.ShapeDtypeStruct((L, NB, D), jnp.float32),
        mesh=mesh,
        scratch_types=[
            pltpu.VMEM((L, CB), jnp.int32),
        ] + [pltpu.VMEM((CB, D), jnp.float32) for _ in range(NBUF)] + [
            pltpu.SemaphoreType.DMA((NBUF,)),
            pltpu.SemaphoreType.DMA((NBUF,)),
        ],
    )
    def gather_kernel(table_hbm, idx_hbm, out_hbm, idx_all, *bufs_and_sems):
        rows = list(bufs_and_sems[:NBUF])
        gsem, osem = bufs_and_sems[NBUF], bufs_and_sems[NBUF + 1]
        wid = lax.axis_index("s") * NC + lax.axis_index("c")

        pltpu.sync_copy(idx_hbm.at[:, wid], idx_all)

        def start_gather(g, b):
            pltpu.make_async_copy(
                table_hbm.at[idx_all.at[g]], rows[b], gsem.at[b]
            ).start()

        def wait_gather(g, b):
            pltpu.make_async_copy(
                table_hbm.at[idx_all.at[g]], rows[b], gsem.at[b]
            ).wait()

        def out_copy(g, b):
            return pltpu.make_async_copy(
                rows[b], out_hbm.at[g, pl.ds(wid * CB, CB)], osem.at[b]
            )

        for b in range(NBUF - 1):
            start_gather(b, b)

        def outer(o, _):
            for b in range(NBUF):
                g = o * NBUF + b
                bb = (b + NBUF - 1) % NBUF
                wait_gather(g, b)

                def group_body(gr, _):
                    iv = idx_all[g, pl.ds(gr * 16, 16)]
                    scv = jnp.where(iv == 0, 0.0, SQRT_D)
                    for r in range(16):
                        j = gr * 16 + r
                        sc = scv[r]
                        for k in range(D // 16):
                            sl = pl.ds(k * 16, 16)
                            rows[b][j, sl] = rows[b][j, sl] * sc
                    return 0

                lax.fori_loop(0, CB // 16, group_body, 0, unroll=False)

                out_copy(g, b).start()

                nxt = g + NBUF - 1

                @pl.when(jnp.logical_and(g >= 1, nxt < L))
                def _():
                    out_copy(g - 1, bb).wait()

                @pl.when(nxt < L)
                def _():
                    start_gather(nxt, bb)
            return 0

        lax.fori_loop(0, L // NBUF, outer, 0, unroll=False)

        for b in range(NBUF):
            out_copy(L - NBUF + b, b).wait()

    return gather_kernel


def kernel(x, table):
    NB, L = x.shape
    V, D = table.shape
    xt = x.T.reshape(L, NB // CB, CB).astype(jnp.int32)
    out = _make_gather(NB, L, V, D)(table, xt)
    return out.transpose(1, 0, 2)
